# Initial kernel scaffold; baseline (speedup 1.0000x reference)
import jax
import jax.numpy as jnp
from jax.experimental import pallas as pl


def kernel(feat_table, nodes, neigh_idx):
    raise NotImplementedError("write your pallas kernel here")


# speedup vs baseline: 3.9566x; 3.9566x over previous
import functools

import jax
import jax.numpy as jnp
from jax import lax
from jax.experimental import pallas as pl
from jax.experimental.pallas import tpu as pltpu
from jax.experimental.pallas import tpu_sc as plsc

N_NODES = 100000
D = 128
BATCH = 50000
S = 10
L = 16

NC = 2
NS = 16
NW = NC * NS

B_PER_W = 1568
BP = NW * B_PER_W
C = 56
NCH = B_PER_W // C
NIDX_SPLIT = 5
NIDX_PER = C * S // NIDX_SPLIT


def _sc_encoder(feat_table, nodes_pad, neigh_flat):
    mesh = plsc.VectorSubcoreMesh(core_axis_name="c", subcore_axis_name="s")

    @functools.partial(
        pl.kernel,
        mesh=mesh,
        out_type=jax.ShapeDtypeStruct((BP, 2 * D), jnp.float32),
        scratch_types=[
            pltpu.VMEM((B_PER_W,), jnp.int32),
            pltpu.VMEM((B_PER_W * S,), jnp.int32),
            pltpu.VMEM((C, D), jnp.float32),
            pltpu.VMEM((C * S, D), jnp.float32),
            pltpu.VMEM((C, D), jnp.float32),
            pltpu.SemaphoreType.DMA,
        ],
    )
    def body(table_h, nodes_h, neigh_h, out_h, sidx, nidx, srows, nrows, mrows, sem):
        wid = lax.axis_index("s") * NC + lax.axis_index("c")
        base = wid * B_PER_W
        pltpu.sync_copy(nodes_h.at[pl.ds(base, B_PER_W)], sidx)
        pltpu.sync_copy(neigh_h.at[pl.ds(base * S, B_PER_W * S)], nidx)

        def chunk(ci, carry):
            off = ci * C
            copies = [
                pltpu.async_copy(table_h.at[sidx.at[pl.ds(off, C)]], srows, sem)
            ]
            for k in range(NIDX_SPLIT):
                copies.append(
                    pltpu.async_copy(
                        table_h.at[nidx.at[pl.ds(off * S + k * NIDX_PER, NIDX_PER)]],
                        nrows.at[pl.ds(k * NIDX_PER, NIDX_PER)],
                        sem,
                    )
                )
            for cp in copies:
                cp.wait()

            def row(r, carry2):
                rb = r * S
                for g in range(D // L):
                    sl = pl.ds(g * L, L)
                    acc = nrows[rb, sl]
                    for j in range(1, S):
                        acc = acc + nrows[rb + j, sl]
                    mrows[r, sl] = acc * jnp.float32(1.0 / S)
                return carry2

            lax.fori_loop(0, C, row, 0, unroll=False)

            row0 = base + off
            pltpu.sync_copy(srows, out_h.at[pl.ds(row0, C), pl.ds(0, D)])
            pltpu.sync_copy(mrows, out_h.at[pl.ds(row0, C), pl.ds(D, D)])
            return carry

        lax.fori_loop(0, NCH, chunk, 0, unroll=False)

    return body(feat_table, nodes_pad, neigh_flat)


@jax.jit
def kernel(feat_table, nodes, neigh_idx):
    nodes_pad = jnp.pad(nodes, (0, BP - BATCH))
    neigh_flat = jnp.pad(neigh_idx.reshape(-1), (0, (BP - BATCH) * S))
    out = _sc_encoder(feat_table, nodes_pad, neigh_flat)
    return out[:BATCH]
